# Initial kernel scaffold; baseline (speedup 1.0000x reference)
import jax
import jax.numpy as jnp
from jax.experimental import pallas as pl


def kernel(x, adj, W1, b1, W2, b2, Wd, bd, We2d, mask_token):
    raise NotImplementedError("write your pallas kernel here")


# speedup vs baseline: 6.2426x; 6.2426x over previous
import functools

import jax
import jax.numpy as jnp
from jax import lax
from jax.experimental import pallas as pl
from jax.experimental.pallas import tpu as pltpu
from jax.experimental.pallas import tpu_sc as plsc

N = 10000
NP = 10240
E = 320000
D_IN = 128
D_H = 256
NUM_MASK = 2500

_SC_MESH = plsc.VectorSubcoreMesh(core_axis_name="c", subcore_axis_name="s")
NUM_CORES = 2
NUM_SUBCORES = 16
NUM_WORKERS = NUM_CORES * NUM_SUBCORES

CHUNK = 80
ROWS_PER_TEC = NP // NUM_SUBCORES
ZROWS = 128
_SC_PARAMS = pltpu.CompilerParams(needs_layout_passes=False)


def _mask_indicator():
    perm = jax.random.permutation(jax.random.key(1), N)
    return jnp.zeros((NP, 1), jnp.float32).at[perm[:NUM_MASK], 0].set(1.0)


def _deg_kernel(dst):
    @functools.partial(
        pl.kernel,
        out_type=jax.ShapeDtypeStruct((NUM_WORKERS, NP), jnp.float32),
        mesh=_SC_MESH,
        scratch_types=[
            pltpu.VMEM((E // NUM_WORKERS,), jnp.int32),
            pltpu.VMEM((NP,), jnp.float32),
        ],
        compiler_params=_SC_PARAMS,
    )
    def k(dst_hbm, hist_out, idx_v, hist_v):
        c = lax.axis_index("c")
        s = lax.axis_index("s")
        wid = c * NUM_SUBCORES + s
        per_w = E // NUM_WORKERS
        ones16 = jnp.ones((16,), jnp.float32)

        def zero(i, _):
            hist_v[pl.ds(i * 16, 16)] = jnp.zeros((16,), jnp.float32)
            return 0

        lax.fori_loop(0, NP // 16, zero, 0)
        pltpu.sync_copy(dst_hbm.at[pl.ds(wid * per_w, per_w)], idx_v)

        def upd(i, _):
            v = idx_v[pl.ds(i * 16, 16)]
            plsc.addupdate_scatter(hist_v, [v], ones16)
            return 0

        lax.fori_loop(0, per_w // 16, upd, 0)
        pltpu.sync_copy(hist_v, hist_out.at[wid])

    return k(dst)


def _scatter_core(s, edge_base, edges_per_tec, src_hbm, dst_hbm, tbl_hbm,
                  out_hbm, acc, zbuf, rows, obuf, sidx, didx, sem):
    for j in range(ROWS_PER_TEC // ZROWS):
        pltpu.sync_copy(zbuf, acc.at[pl.ds(s * ROWS_PER_TEC + j * ZROWS, ZROWS)])
    plsc.subcore_barrier()

    my_base = edge_base + s * edges_per_tec

    def chunk(j, _):
        off = my_base + j * CHUNK
        pltpu.sync_copy(src_hbm.at[pl.ds(off, CHUNK)], sidx)
        pltpu.sync_copy(dst_hbm.at[pl.ds(off, CHUNK)], didx)
        pltpu.async_copy(tbl_hbm.at[sidx], rows, sem).wait()
        pltpu.sync_copy(rows, acc.at[didx], add=True)
        return 0

    lax.fori_loop(0, edges_per_tec // CHUNK, chunk, 0)
    plsc.subcore_barrier()

    for j in range(ROWS_PER_TEC // ZROWS):
        r0 = s * ROWS_PER_TEC + j * ZROWS
        pltpu.sync_copy(acc.at[pl.ds(r0, ZROWS)], obuf)
        pltpu.sync_copy(obuf, out_hbm.at[pl.ds(r0, ZROWS)])


_SCATTER_SCRATCH = [
    pltpu.VMEM_SHARED((NP, 128), jnp.float32),
    pltpu.VMEM((ZROWS, 128), jnp.float32),
    pltpu.VMEM((CHUNK, 128), jnp.float32),
    pltpu.VMEM((ZROWS, 128), jnp.float32),
    pltpu.VMEM((CHUNK,), jnp.int32),
    pltpu.VMEM((CHUNK,), jnp.int32),
    pltpu.SemaphoreType.DMA,
]

_SCATTER_OUT = (
    jax.ShapeDtypeStruct((NP, 128), jnp.float32),
    jax.ShapeDtypeStruct((NP, 128), jnp.float32),
)


def _scatter256(src, dst, ta, tb, zeros_chunk):
    @functools.partial(
        pl.kernel,
        out_type=_SCATTER_OUT,
        mesh=_SC_MESH,
        scratch_types=_SCATTER_SCRATCH,
        compiler_params=_SC_PARAMS,
    )
    def k(src_hbm, dst_hbm, ta_hbm, tb_hbm, z_hbm, sa_hbm, sb_hbm,
          acc, zbuf, rows, obuf, sidx, didx, sem):
        c = lax.axis_index("c")
        s = lax.axis_index("s")
        pltpu.sync_copy(z_hbm, zbuf)
        per_tec = E // NUM_SUBCORES

        @pl.when(c == 0)
        def _():
            _scatter_core(s, 0, per_tec, src_hbm, dst_hbm, ta_hbm, sa_hbm,
                          acc, zbuf, rows, obuf, sidx, didx, sem)

        @pl.when(c == 1)
        def _():
            _scatter_core(s, 0, per_tec, src_hbm, dst_hbm, tb_hbm, sb_hbm,
                          acc, zbuf, rows, obuf, sidx, didx, sem)

    return k(src, dst, ta, tb, zeros_chunk)


def _scatter128(src, dst, t, zeros_chunk):
    @functools.partial(
        pl.kernel,
        out_type=_SCATTER_OUT,
        mesh=_SC_MESH,
        scratch_types=_SCATTER_SCRATCH,
        compiler_params=_SC_PARAMS,
    )
    def k(src_hbm, dst_hbm, t_hbm, z_hbm, s0_hbm, s1_hbm,
          acc, zbuf, rows, obuf, sidx, didx, sem):
        c = lax.axis_index("c")
        s = lax.axis_index("s")
        pltpu.sync_copy(z_hbm, zbuf)
        per_tec = (E // NUM_CORES) // NUM_SUBCORES

        @pl.when(c == 0)
        def _():
            _scatter_core(s, 0, per_tec, src_hbm, dst_hbm, t_hbm, s0_hbm,
                          acc, zbuf, rows, obuf, sidx, didx, sem)

        @pl.when(c == 1)
        def _():
            _scatter_core(s, E // NUM_CORES, per_tec, src_hbm, dst_hbm, t_hbm,
                          s1_hbm, acc, zbuf, rows, obuf, sidx, didx, sem)

    return k(src, dst, t, zeros_chunk)


BLK = 1024


def _invk_body(hist_ref, inv_ref):
    deg = 1.0 + jnp.sum(hist_ref[...], axis=0)
    inv_ref[...] = lax.rsqrt(deg)[:, None]


def _invk(hist):
    return pl.pallas_call(
        _invk_body,
        grid=(1,),
        in_specs=[pl.BlockSpec((NUM_WORKERS, NP), lambda i: (0, 0))],
        out_specs=pl.BlockSpec((NP, 1), lambda i: (0, 0)),
        out_shape=jax.ShapeDtypeStruct((NP, 1), jnp.float32),
    )(hist)


def _prep1_body(inv_ref, x_ref, mf_ref, tok_ref, w1_ref, oa_ref, ob_ref):
    inv = inv_ref[...]
    mf = mf_ref[...]
    out_x = x_ref[...] * (1.0 - mf) + mf * tok_ref[...]
    h1 = jnp.dot(out_x, w1_ref[...], preferred_element_type=jnp.float32)
    h1p = inv * h1
    oa_ref[...] = h1p[:, :128]
    ob_ref[...] = h1p[:, 128:]


def _prep1(inv, x, maskf, tok, w1):
    return pl.pallas_call(
        _prep1_body,
        grid=(NP // BLK,),
        in_specs=[
            pl.BlockSpec((BLK, 1), lambda i: (i, 0)),
            pl.BlockSpec((BLK, D_IN), lambda i: (i, 0)),
            pl.BlockSpec((BLK, 1), lambda i: (i, 0)),
            pl.BlockSpec((1, D_IN), lambda i: (0, 0)),
            pl.BlockSpec((D_IN, D_H), lambda i: (0, 0)),
        ],
        out_specs=[
            pl.BlockSpec((BLK, 128), lambda i: (i, 0)),
            pl.BlockSpec((BLK, 128), lambda i: (i, 0)),
        ],
        out_shape=[
            jax.ShapeDtypeStruct((NP, 128), jnp.float32),
            jax.ShapeDtypeStruct((NP, 128), jnp.float32),
        ],
    )(inv, x, maskf, tok, w1)


def _mid_body(inv_ref, sa_ref, sb_ref, ha_ref, hb_ref, b1_ref, w2_ref,
              oa_ref, ob_ref):
    inv = inv_ref[...]
    b1 = b1_ref[...]
    o_a = inv * (sa_ref[...] + ha_ref[...]) + b1[:, :128]
    o_b = inv * (sb_ref[...] + hb_ref[...]) + b1[:, 128:]
    w2 = w2_ref[...]
    h2 = (jnp.dot(o_a, w2[:128, :], preferred_element_type=jnp.float32)
          + jnp.dot(o_b, w2[128:, :], preferred_element_type=jnp.float32))
    h2p = inv * h2
    oa_ref[...] = h2p[:, :128]
    ob_ref[...] = h2p[:, 128:]


def _mid(inv, sa, sb, ha, hb, b1, w2):
    return pl.pallas_call(
        _mid_body,
        grid=(NP // BLK,),
        in_specs=[
            pl.BlockSpec((BLK, 1), lambda i: (i, 0)),
            pl.BlockSpec((BLK, 128), lambda i: (i, 0)),
            pl.BlockSpec((BLK, 128), lambda i: (i, 0)),
            pl.BlockSpec((BLK, 128), lambda i: (i, 0)),
            pl.BlockSpec((BLK, 128), lambda i: (i, 0)),
            pl.BlockSpec((1, D_H), lambda i: (0, 0)),
            pl.BlockSpec((D_H, D_H), lambda i: (0, 0)),
        ],
        out_specs=[
            pl.BlockSpec((BLK, 128), lambda i: (i, 0)),
            pl.BlockSpec((BLK, 128), lambda i: (i, 0)),
        ],
        out_shape=[
            jax.ShapeDtypeStruct((NP, 128), jnp.float32),
            jax.ShapeDtypeStruct((NP, 128), jnp.float32),
        ],
    )(inv, sa, sb, ha, hb, b1, w2)


def _dec_body(inv_ref, sa_ref, sb_ref, ha_ref, hb_ref, b2_ref, we_ref,
              wd_ref, nmf_ref, o_ref):
    inv = inv_ref[...]
    b2 = b2_ref[...]
    e_a = inv * (sa_ref[...] + ha_ref[...]) + b2[:, :128]
    e_b = inv * (sb_ref[...] + hb_ref[...]) + b2[:, 128:]
    we = we_ref[...]
    rep = (jnp.dot(e_a, we[:128, :], preferred_element_type=jnp.float32)
           + jnp.dot(e_b, we[128:, :], preferred_element_type=jnp.float32))
    rep = rep * nmf_ref[...]
    h3 = jnp.dot(rep, wd_ref[...], preferred_element_type=jnp.float32)
    o_ref[...] = inv * h3


def _dec(inv, sa, sb, ha, hb, b2, we2d, wd, nmaskf):
    return pl.pallas_call(
        _dec_body,
        grid=(NP // BLK,),
        in_specs=[
            pl.BlockSpec((BLK, 1), lambda i: (i, 0)),
            pl.BlockSpec((BLK, 128), lambda i: (i, 0)),
            pl.BlockSpec((BLK, 128), lambda i: (i, 0)),
            pl.BlockSpec((BLK, 128), lambda i: (i, 0)),
            pl.BlockSpec((BLK, 128), lambda i: (i, 0)),
            pl.BlockSpec((1, D_H), lambda i: (0, 0)),
            pl.BlockSpec((D_H, D_H), lambda i: (0, 0)),
            pl.BlockSpec((D_H, D_IN), lambda i: (0, 0)),
            pl.BlockSpec((BLK, 1), lambda i: (i, 0)),
        ],
        out_specs=pl.BlockSpec((BLK, D_IN), lambda i: (i, 0)),
        out_shape=jax.ShapeDtypeStruct((NP, D_IN), jnp.float32),
    )(inv, sa, sb, ha, hb, b2, we2d, wd, nmaskf)


def _loss_body(inv_ref, x_ref, s0_ref, s1_ref, h3_ref, bd_ref, mf_ref,
               o_ref):
    i = pl.program_id(0)
    inv = inv_ref[...]
    recon = inv * (s0_ref[...] + s1_ref[...] + h3_ref[...]) + bd_ref[...]
    d = (x_ref[...] - recon) * mf_ref[...]
    part = jnp.sum(d * d)

    @pl.when(i == 0)
    def _():
        o_ref[...] = jnp.zeros_like(o_ref)

    o_ref[...] += part[None, None]


def _loss(inv, x, s0, s1, h3p, bd, maskf):
    return pl.pallas_call(
        _loss_body,
        grid=(NP // BLK,),
        in_specs=[
            pl.BlockSpec((BLK, 1), lambda i: (i, 0)),
            pl.BlockSpec((BLK, D_IN), lambda i: (i, 0)),
            pl.BlockSpec((BLK, D_IN), lambda i: (i, 0)),
            pl.BlockSpec((BLK, D_IN), lambda i: (i, 0)),
            pl.BlockSpec((BLK, D_IN), lambda i: (i, 0)),
            pl.BlockSpec((1, D_IN), lambda i: (0, 0)),
            pl.BlockSpec((BLK, 1), lambda i: (i, 0)),
        ],
        out_specs=pl.BlockSpec((1, 1), lambda i: (0, 0)),
        out_shape=jax.ShapeDtypeStruct((1, 1), jnp.float32),
    )(inv, x, s0, s1, h3p, bd, maskf)


def kernel(x, adj, W1, b1, W2, b2, Wd, bd, We2d, mask_token):
    src = adj[0]
    dst = adj[1]
    xp = jnp.pad(x, ((0, NP - N), (0, 0)))
    maskf = _mask_indicator()
    nmaskf = 1.0 - maskf
    zeros_chunk = jnp.zeros((ZROWS, 128), jnp.float32)
    b1r = b1.reshape(1, D_H)
    b2r = b2.reshape(1, D_H)
    bdr = bd.reshape(1, D_IN)

    hist = _deg_kernel(dst)
    inv = _invk(hist)
    h1a, h1b = _prep1(inv, xp, maskf, mask_token, W1)
    s1a, s1b = _scatter256(src, dst, h1a, h1b, zeros_chunk)
    h2a, h2b = _mid(inv, s1a, s1b, h1a, h1b, b1r, W2)
    s2a, s2b = _scatter256(src, dst, h2a, h2b, zeros_chunk)
    h3p = _dec(inv, s2a, s2b, h2a, h2b, b2r, We2d, Wd, nmaskf)
    s30, s31 = _scatter128(src, dst, h3p, zeros_chunk)
    losssum = _loss(inv, xp, s30, s31, h3p, bdr, maskf)
    return losssum[0, 0] * (1.0 / (NUM_MASK * D_IN))


# speedup vs baseline: 14.6975x; 2.3544x over previous
import functools

import jax
import jax.numpy as jnp
from jax import lax
from jax.experimental import pallas as pl
from jax.experimental.pallas import tpu as pltpu
from jax.experimental.pallas import tpu_sc as plsc

N = 10000
NP = 10240
E = 320000
D_IN = 128
D_H = 256
NUM_MASK = 2500

_SC_MESH = plsc.VectorSubcoreMesh(core_axis_name="c", subcore_axis_name="s")
NUM_CORES = 2
NUM_SUBCORES = 16
NUM_WORKERS = NUM_CORES * NUM_SUBCORES

CHUNK = 125
IDXBLK = 16
ROWS_PER_TEC = NP // NUM_SUBCORES
ZROWS = 32
_SC_PARAMS = pltpu.CompilerParams(needs_layout_passes=False)


def _mask_indicator():
    perm = jax.random.permutation(jax.random.key(1), N)
    return jnp.zeros((NP, 1), jnp.float32).at[perm[:NUM_MASK], 0].set(1.0)


def _deg_kernel(dst):
    @functools.partial(
        pl.kernel,
        out_type=jax.ShapeDtypeStruct((NUM_WORKERS, NP), jnp.float32),
        mesh=_SC_MESH,
        scratch_types=[
            pltpu.VMEM((E // NUM_WORKERS,), jnp.int32),
            pltpu.VMEM((NP,), jnp.float32),
        ],
        compiler_params=_SC_PARAMS,
    )
    def k(dst_hbm, hist_out, idx_v, hist_v):
        c = lax.axis_index("c")
        s = lax.axis_index("s")
        wid = c * NUM_SUBCORES + s
        per_w = E // NUM_WORKERS
        ones16 = jnp.ones((16,), jnp.float32)

        def zero(i, _):
            hist_v[pl.ds(i * 16, 16)] = jnp.zeros((16,), jnp.float32)
            return 0

        lax.fori_loop(0, NP // 16, zero, 0)
        pltpu.sync_copy(dst_hbm.at[pl.ds(wid * per_w, per_w)], idx_v)

        def upd(i, _):
            v = idx_v[pl.ds(i * 16, 16)]
            plsc.addupdate_scatter(hist_v, [v], ones16)
            return 0

        lax.fori_loop(0, per_w // 16, upd, 0)
        pltpu.sync_copy(hist_v, hist_out.at[wid])

    return k(dst)


def _scatter_core(s, chunk_base, nchunks, src_hbm, dst_hbm, tbl_hbm,
                  out_hbm, acc, stage, rows0, rows1, sidx, didx, sem):
    for j in range(ROWS_PER_TEC // ZROWS):
        pltpu.sync_copy(stage, acc.at[pl.ds(s * ROWS_PER_TEC + j * ZROWS, ZROWS)])
    plsc.subcore_barrier()

    row0 = chunk_base + s * nchunks

    def block(b, _):
        pltpu.sync_copy(src_hbm.at[pl.ds(row0 + b * IDXBLK, IDXBLK)], sidx)
        pltpu.sync_copy(dst_hbm.at[pl.ds(row0 + b * IDXBLK, IDXBLK)], didx)
        pltpu.async_copy(tbl_hbm.at[sidx.at[0]], rows0, sem)

        def pair(jj, last):
            j0 = 2 * jj
            d1 = pltpu.async_copy(tbl_hbm.at[sidx.at[j0 + 1]], rows1, sem)
            pltpu.make_async_copy(tbl_hbm.at[sidx.at[j0]], rows0, sem).wait()
            pltpu.sync_copy(rows0, acc.at[didx.at[j0]], add=True)
            if not last:
                pltpu.async_copy(tbl_hbm.at[sidx.at[j0 + 2]], rows0, sem)
            d1.wait()
            pltpu.sync_copy(rows1, acc.at[didx.at[j0 + 1]], add=True)

        def body(jj, _):
            pair(jj, last=False)
            return 0

        lax.fori_loop(0, IDXBLK // 2 - 1, body, 0)
        pair(IDXBLK // 2 - 1, last=True)
        return 0

    lax.fori_loop(0, nchunks // IDXBLK, block, 0)
    plsc.subcore_barrier()

    for j in range(ROWS_PER_TEC // ZROWS):
        r0 = s * ROWS_PER_TEC + j * ZROWS
        pltpu.sync_copy(acc.at[pl.ds(r0, ZROWS)], stage)
        pltpu.sync_copy(stage, out_hbm.at[pl.ds(r0, ZROWS)])


_SCATTER_SCRATCH = [
    pltpu.VMEM_SHARED((NP, 128), jnp.float32),
    pltpu.VMEM((ZROWS, 128), jnp.float32),
    pltpu.VMEM((CHUNK, 128), jnp.float32),
    pltpu.VMEM((CHUNK, 128), jnp.float32),
    pltpu.VMEM((IDXBLK, CHUNK), jnp.int32),
    pltpu.VMEM((IDXBLK, CHUNK), jnp.int32),
    pltpu.SemaphoreType.DMA,
]

_SCATTER_OUT = (
    jax.ShapeDtypeStruct((NP, 128), jnp.float32),
    jax.ShapeDtypeStruct((NP, 128), jnp.float32),
)


def _scatter256(src, dst, ta, tb, zeros_chunk):
    @functools.partial(
        pl.kernel,
        out_type=_SCATTER_OUT,
        mesh=_SC_MESH,
        scratch_types=_SCATTER_SCRATCH,
        compiler_params=_SC_PARAMS,
    )
    def k(src_hbm, dst_hbm, ta_hbm, tb_hbm, z_hbm, sa_hbm, sb_hbm,
          acc, stage, rows0, rows1, sidx, didx, sem):
        c = lax.axis_index("c")
        s = lax.axis_index("s")
        pltpu.sync_copy(z_hbm, stage)
        nchunks = E // CHUNK // NUM_SUBCORES

        @pl.when(c == 0)
        def _():
            _scatter_core(s, 0, nchunks, src_hbm, dst_hbm, ta_hbm, sa_hbm,
                          acc, stage, rows0, rows1, sidx, didx, sem)

        @pl.when(c == 1)
        def _():
            _scatter_core(s, 0, nchunks, src_hbm, dst_hbm, tb_hbm, sb_hbm,
                          acc, stage, rows0, rows1, sidx, didx, sem)

    return k(src, dst, ta, tb, zeros_chunk)


def _scatter128(src, dst, t, zeros_chunk):
    @functools.partial(
        pl.kernel,
        out_type=_SCATTER_OUT,
        mesh=_SC_MESH,
        scratch_types=_SCATTER_SCRATCH,
        compiler_params=_SC_PARAMS,
    )
    def k(src_hbm, dst_hbm, t_hbm, z_hbm, s0_hbm, s1_hbm,
          acc, stage, rows0, rows1, sidx, didx, sem):
        c = lax.axis_index("c")
        s = lax.axis_index("s")
        pltpu.sync_copy(z_hbm, stage)
        nchunks = E // CHUNK // NUM_CORES // NUM_SUBCORES
        half_rows = E // CHUNK // NUM_CORES

        @pl.when(c == 0)
        def _():
            _scatter_core(s, 0, nchunks, src_hbm, dst_hbm, t_hbm, s0_hbm,
                          acc, stage, rows0, rows1, sidx, didx, sem)

        @pl.when(c == 1)
        def _():
            _scatter_core(s, half_rows, nchunks, src_hbm, dst_hbm, t_hbm,
                          s1_hbm, acc, stage, rows0, rows1, sidx, didx, sem)

    return k(src, dst, t, zeros_chunk)


BLK = 1024


def _invk_body(hist_ref, inv_ref):
    deg = 1.0 + jnp.sum(hist_ref[...], axis=0)
    inv_ref[...] = lax.rsqrt(deg)[:, None]


def _invk(hist):
    return pl.pallas_call(
        _invk_body,
        grid=(1,),
        in_specs=[pl.BlockSpec((NUM_WORKERS, NP), lambda i: (0, 0))],
        out_specs=pl.BlockSpec((NP, 1), lambda i: (0, 0)),
        out_shape=jax.ShapeDtypeStruct((NP, 1), jnp.float32),
    )(hist)


def _prep1_body(inv_ref, x_ref, mf_ref, tok_ref, w1_ref, oa_ref, ob_ref):
    inv = inv_ref[...]
    mf = mf_ref[...]
    out_x = x_ref[...] * (1.0 - mf) + mf * tok_ref[...]
    h1 = jnp.dot(out_x, w1_ref[...], preferred_element_type=jnp.float32)
    h1p = inv * h1
    oa_ref[...] = h1p[:, :128]
    ob_ref[...] = h1p[:, 128:]


def _prep1(inv, x, maskf, tok, w1):
    return pl.pallas_call(
        _prep1_body,
        grid=(NP // BLK,),
        in_specs=[
            pl.BlockSpec((BLK, 1), lambda i: (i, 0)),
            pl.BlockSpec((BLK, D_IN), lambda i: (i, 0)),
            pl.BlockSpec((BLK, 1), lambda i: (i, 0)),
            pl.BlockSpec((1, D_IN), lambda i: (0, 0)),
            pl.BlockSpec((D_IN, D_H), lambda i: (0, 0)),
        ],
        out_specs=[
            pl.BlockSpec((BLK, 128), lambda i: (i, 0)),
            pl.BlockSpec((BLK, 128), lambda i: (i, 0)),
        ],
        out_shape=[
            jax.ShapeDtypeStruct((NP, 128), jnp.float32),
            jax.ShapeDtypeStruct((NP, 128), jnp.float32),
        ],
    )(inv, x, maskf, tok, w1)


def _mid_body(inv_ref, sa_ref, sb_ref, ha_ref, hb_ref, b1_ref, w2_ref,
              oa_ref, ob_ref):
    inv = inv_ref[...]
    b1 = b1_ref[...]
    o_a = inv * (sa_ref[...] + ha_ref[...]) + b1[:, :128]
    o_b = inv * (sb_ref[...] + hb_ref[...]) + b1[:, 128:]
    w2 = w2_ref[...]
    h2 = (jnp.dot(o_a, w2[:128, :], preferred_element_type=jnp.float32)
          + jnp.dot(o_b, w2[128:, :], preferred_element_type=jnp.float32))
    h2p = inv * h2
    oa_ref[...] = h2p[:, :128]
    ob_ref[...] = h2p[:, 128:]


def _mid(inv, sa, sb, ha, hb, b1, w2):
    return pl.pallas_call(
        _mid_body,
        grid=(NP // BLK,),
        in_specs=[
            pl.BlockSpec((BLK, 1), lambda i: (i, 0)),
            pl.BlockSpec((BLK, 128), lambda i: (i, 0)),
            pl.BlockSpec((BLK, 128), lambda i: (i, 0)),
            pl.BlockSpec((BLK, 128), lambda i: (i, 0)),
            pl.BlockSpec((BLK, 128), lambda i: (i, 0)),
            pl.BlockSpec((1, D_H), lambda i: (0, 0)),
            pl.BlockSpec((D_H, D_H), lambda i: (0, 0)),
        ],
        out_specs=[
            pl.BlockSpec((BLK, 128), lambda i: (i, 0)),
            pl.BlockSpec((BLK, 128), lambda i: (i, 0)),
        ],
        out_shape=[
            jax.ShapeDtypeStruct((NP, 128), jnp.float32),
            jax.ShapeDtypeStruct((NP, 128), jnp.float32),
        ],
    )(inv, sa, sb, ha, hb, b1, w2)


def _dec_body(inv_ref, sa_ref, sb_ref, ha_ref, hb_ref, b2_ref, we_ref,
              wd_ref, nmf_ref, o_ref):
    inv = inv_ref[...]
    b2 = b2_ref[...]
    e_a = inv * (sa_ref[...] + ha_ref[...]) + b2[:, :128]
    e_b = inv * (sb_ref[...] + hb_ref[...]) + b2[:, 128:]
    we = we_ref[...]
    rep = (jnp.dot(e_a, we[:128, :], preferred_element_type=jnp.float32)
           + jnp.dot(e_b, we[128:, :], preferred_element_type=jnp.float32))
    rep = rep * nmf_ref[...]
    h3 = jnp.dot(rep, wd_ref[...], preferred_element_type=jnp.float32)
    o_ref[...] = inv * h3


def _dec(inv, sa, sb, ha, hb, b2, we2d, wd, nmaskf):
    return pl.pallas_call(
        _dec_body,
        grid=(NP // BLK,),
        in_specs=[
            pl.BlockSpec((BLK, 1), lambda i: (i, 0)),
            pl.BlockSpec((BLK, 128), lambda i: (i, 0)),
            pl.BlockSpec((BLK, 128), lambda i: (i, 0)),
            pl.BlockSpec((BLK, 128), lambda i: (i, 0)),
            pl.BlockSpec((BLK, 128), lambda i: (i, 0)),
            pl.BlockSpec((1, D_H), lambda i: (0, 0)),
            pl.BlockSpec((D_H, D_H), lambda i: (0, 0)),
            pl.BlockSpec((D_H, D_IN), lambda i: (0, 0)),
            pl.BlockSpec((BLK, 1), lambda i: (i, 0)),
        ],
        out_specs=pl.BlockSpec((BLK, D_IN), lambda i: (i, 0)),
        out_shape=jax.ShapeDtypeStruct((NP, D_IN), jnp.float32),
    )(inv, sa, sb, ha, hb, b2, we2d, wd, nmaskf)


def _loss_body(inv_ref, x_ref, s0_ref, s1_ref, h3_ref, bd_ref, mf_ref,
               o_ref):
    i = pl.program_id(0)
    inv = inv_ref[...]
    recon = inv * (s0_ref[...] + s1_ref[...] + h3_ref[...]) + bd_ref[...]
    d = (x_ref[...] - recon) * mf_ref[...]
    part = jnp.sum(d * d)

    @pl.when(i == 0)
    def _():
        o_ref[...] = jnp.zeros_like(o_ref)

    o_ref[...] += part[None, None]


def _loss(inv, x, s0, s1, h3p, bd, maskf):
    return pl.pallas_call(
        _loss_body,
        grid=(NP // BLK,),
        in_specs=[
            pl.BlockSpec((BLK, 1), lambda i: (i, 0)),
            pl.BlockSpec((BLK, D_IN), lambda i: (i, 0)),
            pl.BlockSpec((BLK, D_IN), lambda i: (i, 0)),
            pl.BlockSpec((BLK, D_IN), lambda i: (i, 0)),
            pl.BlockSpec((BLK, D_IN), lambda i: (i, 0)),
            pl.BlockSpec((1, D_IN), lambda i: (0, 0)),
            pl.BlockSpec((BLK, 1), lambda i: (i, 0)),
        ],
        out_specs=pl.BlockSpec((1, 1), lambda i: (0, 0)),
        out_shape=jax.ShapeDtypeStruct((1, 1), jnp.float32),
    )(inv, x, s0, s1, h3p, bd, maskf)


def kernel(x, adj, W1, b1, W2, b2, Wd, bd, We2d, mask_token):
    src = adj[0].reshape(E // CHUNK, CHUNK)
    dst = adj[1].reshape(E // CHUNK, CHUNK)
    xp = jnp.pad(x, ((0, NP - N), (0, 0)))
    maskf = _mask_indicator()
    nmaskf = 1.0 - maskf
    zeros_chunk = jnp.zeros((ZROWS, 128), jnp.float32)
    b1r = b1.reshape(1, D_H)
    b2r = b2.reshape(1, D_H)
    bdr = bd.reshape(1, D_IN)

    hist = _deg_kernel(adj[1])
    inv = _invk(hist)
    h1a, h1b = _prep1(inv, xp, maskf, mask_token, W1)
    s1a, s1b = _scatter256(src, dst, h1a, h1b, zeros_chunk)
    h2a, h2b = _mid(inv, s1a, s1b, h1a, h1b, b1r, W2)
    s2a, s2b = _scatter256(src, dst, h2a, h2b, zeros_chunk)
    h3p = _dec(inv, s2a, s2b, h2a, h2b, b2r, We2d, Wd, nmaskf)
    s30, s31 = _scatter128(src, dst, h3p, zeros_chunk)
    losssum = _loss(inv, xp, s30, s31, h3p, bdr, maskf)
    return losssum[0, 0] * (1.0 / (NUM_MASK * D_IN))


# speedup vs baseline: 14.7888x; 1.0062x over previous
import functools

import jax
import jax.numpy as jnp
from jax import lax
from jax.experimental import pallas as pl
from jax.experimental.pallas import tpu as pltpu
from jax.experimental.pallas import tpu_sc as plsc

N = 10000
NP = 10240
E = 320000
D_IN = 128
D_H = 256
NUM_MASK = 2500

_SC_MESH = plsc.VectorSubcoreMesh(core_axis_name="c", subcore_axis_name="s")
NUM_CORES = 2
NUM_SUBCORES = 16
NUM_WORKERS = NUM_CORES * NUM_SUBCORES

CHUNK = 125
IDXBLK = 16
ROWS_PER_TEC = NP // NUM_SUBCORES
ZROWS = 32
_SC_PARAMS = pltpu.CompilerParams(needs_layout_passes=False)


def _mask_indicator():
    perm = jax.random.permutation(jax.random.key(1), N)
    return jnp.zeros((NP, 1), jnp.float32).at[perm[:NUM_MASK], 0].set(1.0)


def _deg_kernel(dst):
    @functools.partial(
        pl.kernel,
        out_type=jax.ShapeDtypeStruct((NUM_WORKERS, NP), jnp.float32),
        mesh=_SC_MESH,
        scratch_types=[
            pltpu.VMEM((E // NUM_WORKERS,), jnp.int32),
            pltpu.VMEM((NP,), jnp.float32),
        ],
        compiler_params=_SC_PARAMS,
    )
    def k(dst_hbm, hist_out, idx_v, hist_v):
        c = lax.axis_index("c")
        s = lax.axis_index("s")
        wid = c * NUM_SUBCORES + s
        per_w = E // NUM_WORKERS
        ones16 = jnp.ones((16,), jnp.float32)

        def zero(i, _):
            hist_v[pl.ds(i * 16, 16)] = jnp.zeros((16,), jnp.float32)
            return 0

        lax.fori_loop(0, NP // 16, zero, 0)
        pltpu.sync_copy(dst_hbm.at[pl.ds(wid * per_w, per_w)], idx_v)

        def upd(i, _):
            v = idx_v[pl.ds(i * 16, 16)]
            plsc.addupdate_scatter(hist_v, [v], ones16)
            return 0

        lax.fori_loop(0, per_w // 16, upd, 0)
        pltpu.sync_copy(hist_v, hist_out.at[wid])

    return k(dst)


def _scatter_core(s, chunk_base, nchunks, src_hbm, dst_hbm, tbl_hbm,
                  out_hbm, acc, stage, rows0, rows1, sidx, didx, sem):
    for j in range(ROWS_PER_TEC // ZROWS):
        pltpu.sync_copy(stage, acc.at[pl.ds(s * ROWS_PER_TEC + j * ZROWS, ZROWS)])
    plsc.subcore_barrier()

    row0 = chunk_base + s * nchunks

    def block(b, _):
        pltpu.sync_copy(src_hbm.at[pl.ds(row0 + b * IDXBLK, IDXBLK)], sidx)
        pltpu.sync_copy(dst_hbm.at[pl.ds(row0 + b * IDXBLK, IDXBLK)], didx)
        pltpu.async_copy(tbl_hbm.at[sidx.at[0]], rows0, sem)

        def pair(jj, last):
            j0 = 2 * jj
            d1 = pltpu.async_copy(tbl_hbm.at[sidx.at[j0 + 1]], rows1, sem)
            pltpu.make_async_copy(tbl_hbm.at[sidx.at[j0]], rows0, sem).wait()
            pltpu.sync_copy(rows0, acc.at[didx.at[j0]], add=True)
            if not last:
                pltpu.async_copy(tbl_hbm.at[sidx.at[j0 + 2]], rows0, sem)
            d1.wait()
            pltpu.sync_copy(rows1, acc.at[didx.at[j0 + 1]], add=True)

        def body(jj, _):
            pair(jj, last=False)
            return 0

        lax.fori_loop(0, IDXBLK // 2 - 1, body, 0)
        pair(IDXBLK // 2 - 1, last=True)
        return 0

    lax.fori_loop(0, nchunks // IDXBLK, block, 0)
    plsc.subcore_barrier()

    for j in range(ROWS_PER_TEC // ZROWS):
        r0 = s * ROWS_PER_TEC + j * ZROWS
        pltpu.sync_copy(acc.at[pl.ds(r0, ZROWS)], stage)
        pltpu.sync_copy(stage, out_hbm.at[pl.ds(r0, ZROWS)])


_SCATTER_SCRATCH = [
    pltpu.VMEM_SHARED((NP, 128), jnp.float32),
    pltpu.VMEM((ZROWS, 128), jnp.float32),
    pltpu.VMEM((CHUNK, 128), jnp.float32),
    pltpu.VMEM((CHUNK, 128), jnp.float32),
    pltpu.VMEM((IDXBLK, CHUNK), jnp.int32),
    pltpu.VMEM((IDXBLK, CHUNK), jnp.int32),
    pltpu.SemaphoreType.DMA,
]

_SCATTER_OUT = (
    jax.ShapeDtypeStruct((NP, 128), jnp.float32),
    jax.ShapeDtypeStruct((NP, 128), jnp.float32),
)


def _scatter256(src, dst, ta, tb, zeros_chunk):
    @functools.partial(
        pl.kernel,
        out_type=_SCATTER_OUT,
        mesh=_SC_MESH,
        scratch_types=_SCATTER_SCRATCH,
        compiler_params=_SC_PARAMS,
    )
    def k(src_hbm, dst_hbm, ta_hbm, tb_hbm, z_hbm, sa_hbm, sb_hbm,
          acc, stage, rows0, rows1, sidx, didx, sem):
        c = lax.axis_index("c")
        s = lax.axis_index("s")
        pltpu.sync_copy(z_hbm, stage)
        nchunks = E // CHUNK // NUM_SUBCORES

        @pl.when(c == 0)
        def _():
            _scatter_core(s, 0, nchunks, src_hbm, dst_hbm, ta_hbm, sa_hbm,
                          acc, stage, rows0, rows1, sidx, didx, sem)

        @pl.when(c == 1)
        def _():
            _scatter_core(s, 0, nchunks, src_hbm, dst_hbm, tb_hbm, sb_hbm,
                          acc, stage, rows0, rows1, sidx, didx, sem)

    return k(src, dst, ta, tb, zeros_chunk)


def _scatter128(src, dst, t, zeros_chunk):
    @functools.partial(
        pl.kernel,
        out_type=_SCATTER_OUT,
        mesh=_SC_MESH,
        scratch_types=_SCATTER_SCRATCH,
        compiler_params=_SC_PARAMS,
    )
    def k(src_hbm, dst_hbm, t_hbm, z_hbm, s0_hbm, s1_hbm,
          acc, stage, rows0, rows1, sidx, didx, sem):
        c = lax.axis_index("c")
        s = lax.axis_index("s")
        pltpu.sync_copy(z_hbm, stage)
        nchunks = E // CHUNK // NUM_CORES // NUM_SUBCORES
        half_rows = E // CHUNK // NUM_CORES

        @pl.when(c == 0)
        def _():
            _scatter_core(s, 0, nchunks, src_hbm, dst_hbm, t_hbm, s0_hbm,
                          acc, stage, rows0, rows1, sidx, didx, sem)

        @pl.when(c == 1)
        def _():
            _scatter_core(s, half_rows, nchunks, src_hbm, dst_hbm, t_hbm,
                          s1_hbm, acc, stage, rows0, rows1, sidx, didx, sem)

    return k(src, dst, t, zeros_chunk)


BLK = 1024


def _prep1_body(hist_ref, x_ref, mf_ref, tok_ref, w1_ref, oa_ref, ob_ref,
                inv_ref):
    deg = 1.0 + jnp.sum(hist_ref[...], axis=0)
    inv = lax.rsqrt(deg)[:, None]
    inv_ref[...] = inv
    mf = mf_ref[...]
    out_x = x_ref[...] * (1.0 - mf) + mf * tok_ref[...]
    h1 = jnp.dot(out_x.astype(jnp.bfloat16),
                 w1_ref[...].astype(jnp.bfloat16),
                 preferred_element_type=jnp.float32)
    h1p = inv * h1
    oa_ref[...] = h1p[:, :128]
    ob_ref[...] = h1p[:, 128:]


def _prep1(hist, x, maskf, tok, w1):
    return pl.pallas_call(
        _prep1_body,
        grid=(NP // BLK,),
        in_specs=[
            pl.BlockSpec((NUM_WORKERS, BLK), lambda i: (0, i)),
            pl.BlockSpec((BLK, D_IN), lambda i: (i, 0)),
            pl.BlockSpec((BLK, 1), lambda i: (i, 0)),
            pl.BlockSpec((1, D_IN), lambda i: (0, 0)),
            pl.BlockSpec((D_IN, D_H), lambda i: (0, 0)),
        ],
        out_specs=[
            pl.BlockSpec((BLK, 128), lambda i: (i, 0)),
            pl.BlockSpec((BLK, 128), lambda i: (i, 0)),
            pl.BlockSpec((BLK, 1), lambda i: (i, 0)),
        ],
        out_shape=[
            jax.ShapeDtypeStruct((NP, 128), jnp.float32),
            jax.ShapeDtypeStruct((NP, 128), jnp.float32),
            jax.ShapeDtypeStruct((NP, 1), jnp.float32),
        ],
    )(hist, x, maskf, tok, w1)


def _mid_body(inv_ref, sa_ref, sb_ref, ha_ref, hb_ref, b1_ref, w2_ref,
              oa_ref, ob_ref):
    inv = inv_ref[...]
    b1 = b1_ref[...]
    o_a = (inv * (sa_ref[...].astype(jnp.float32)
                  + ha_ref[...].astype(jnp.float32))
           + b1[:, :128]).astype(jnp.bfloat16)
    o_b = (inv * (sb_ref[...].astype(jnp.float32)
                  + hb_ref[...].astype(jnp.float32))
           + b1[:, 128:]).astype(jnp.bfloat16)
    w2 = w2_ref[...].astype(jnp.bfloat16)
    h2 = (jnp.dot(o_a, w2[:128, :], preferred_element_type=jnp.float32)
          + jnp.dot(o_b, w2[128:, :], preferred_element_type=jnp.float32))
    h2p = inv * h2
    oa_ref[...] = h2p[:, :128]
    ob_ref[...] = h2p[:, 128:]


def _mid(inv, sa, sb, ha, hb, b1, w2):
    return pl.pallas_call(
        _mid_body,
        grid=(NP // BLK,),
        in_specs=[
            pl.BlockSpec((BLK, 1), lambda i: (i, 0)),
            pl.BlockSpec((BLK, 128), lambda i: (i, 0)),
            pl.BlockSpec((BLK, 128), lambda i: (i, 0)),
            pl.BlockSpec((BLK, 128), lambda i: (i, 0)),
            pl.BlockSpec((BLK, 128), lambda i: (i, 0)),
            pl.BlockSpec((1, D_H), lambda i: (0, 0)),
            pl.BlockSpec((D_H, D_H), lambda i: (0, 0)),
        ],
        out_specs=[
            pl.BlockSpec((BLK, 128), lambda i: (i, 0)),
            pl.BlockSpec((BLK, 128), lambda i: (i, 0)),
        ],
        out_shape=[
            jax.ShapeDtypeStruct((NP, 128), jnp.float32),
            jax.ShapeDtypeStruct((NP, 128), jnp.float32),
        ],
    )(inv, sa, sb, ha, hb, b1, w2)


def _dec_body(inv_ref, sa_ref, sb_ref, ha_ref, hb_ref, b2_ref, we_ref,
              wd_ref, nmf_ref, o_ref):
    inv = inv_ref[...]
    b2 = b2_ref[...]
    e_a = (inv * (sa_ref[...].astype(jnp.float32)
                  + ha_ref[...].astype(jnp.float32))
           + b2[:, :128]).astype(jnp.bfloat16)
    e_b = (inv * (sb_ref[...].astype(jnp.float32)
                  + hb_ref[...].astype(jnp.float32))
           + b2[:, 128:]).astype(jnp.bfloat16)
    we = we_ref[...].astype(jnp.bfloat16)
    rep = (jnp.dot(e_a, we[:128, :], preferred_element_type=jnp.float32)
           + jnp.dot(e_b, we[128:, :], preferred_element_type=jnp.float32))
    rep = (rep * nmf_ref[...]).astype(jnp.bfloat16)
    h3 = jnp.dot(rep, wd_ref[...].astype(jnp.bfloat16),
                 preferred_element_type=jnp.float32)
    o_ref[...] = inv * h3


def _dec(inv, sa, sb, ha, hb, b2, we2d, wd, nmaskf):
    return pl.pallas_call(
        _dec_body,
        grid=(NP // BLK,),
        in_specs=[
            pl.BlockSpec((BLK, 1), lambda i: (i, 0)),
            pl.BlockSpec((BLK, 128), lambda i: (i, 0)),
            pl.BlockSpec((BLK, 128), lambda i: (i, 0)),
            pl.BlockSpec((BLK, 128), lambda i: (i, 0)),
            pl.BlockSpec((BLK, 128), lambda i: (i, 0)),
            pl.BlockSpec((1, D_H), lambda i: (0, 0)),
            pl.BlockSpec((D_H, D_H), lambda i: (0, 0)),
            pl.BlockSpec((D_H, D_IN), lambda i: (0, 0)),
            pl.BlockSpec((BLK, 1), lambda i: (i, 0)),
        ],
        out_specs=pl.BlockSpec((BLK, D_IN), lambda i: (i, 0)),
        out_shape=jax.ShapeDtypeStruct((NP, D_IN), jnp.float32),
    )(inv, sa, sb, ha, hb, b2, we2d, wd, nmaskf)


def _loss_body(inv_ref, x_ref, s0_ref, s1_ref, h3_ref, bd_ref, mf_ref,
               o_ref):
    i = pl.program_id(0)
    inv = inv_ref[...]
    recon = inv * (s0_ref[...].astype(jnp.float32)
                   + s1_ref[...].astype(jnp.float32)
                   + h3_ref[...].astype(jnp.float32)) + bd_ref[...]
    d = (x_ref[...] - recon) * mf_ref[...]
    part = jnp.sum(d * d)

    @pl.when(i == 0)
    def _():
        o_ref[...] = jnp.zeros_like(o_ref)

    o_ref[...] += part[None, None]


def _loss(inv, x, s0, s1, h3p, bd, maskf):
    return pl.pallas_call(
        _loss_body,
        grid=(NP // BLK,),
        in_specs=[
            pl.BlockSpec((BLK, 1), lambda i: (i, 0)),
            pl.BlockSpec((BLK, D_IN), lambda i: (i, 0)),
            pl.BlockSpec((BLK, D_IN), lambda i: (i, 0)),
            pl.BlockSpec((BLK, D_IN), lambda i: (i, 0)),
            pl.BlockSpec((BLK, D_IN), lambda i: (i, 0)),
            pl.BlockSpec((1, D_IN), lambda i: (0, 0)),
            pl.BlockSpec((BLK, 1), lambda i: (i, 0)),
        ],
        out_specs=pl.BlockSpec((1, 1), lambda i: (0, 0)),
        out_shape=jax.ShapeDtypeStruct((1, 1), jnp.float32),
    )(inv, x, s0, s1, h3p, bd, maskf)


def kernel(x, adj, W1, b1, W2, b2, Wd, bd, We2d, mask_token):
    src = adj[0].reshape(E // CHUNK, CHUNK)
    dst = adj[1].reshape(E // CHUNK, CHUNK)
    xp = jnp.pad(x, ((0, NP - N), (0, 0)))
    maskf = _mask_indicator()
    nmaskf = 1.0 - maskf
    zeros_chunk = jnp.zeros((ZROWS, 128), jnp.float32)
    b1r = b1.reshape(1, D_H)
    b2r = b2.reshape(1, D_H)
    bdr = bd.reshape(1, D_IN)

    hist = _deg_kernel(adj[1])
    h1a, h1b, inv = _prep1(hist, xp, maskf, mask_token, W1)
    s1a, s1b = _scatter256(src, dst, h1a, h1b, zeros_chunk)
    h2a, h2b = _mid(inv, s1a, s1b, h1a, h1b, b1r, W2)
    s2a, s2b = _scatter256(src, dst, h2a, h2b, zeros_chunk)
    h3p = _dec(inv, s2a, s2b, h2a, h2b, b2r, We2d, Wd, nmaskf)
    s30, s31 = _scatter128(src, dst, h3p, zeros_chunk)
    losssum = _loss(inv, xp, s30, s31, h3p, bdr, maskf)
    return losssum[0, 0] * (1.0 / (NUM_MASK * D_IN))
